# Initial kernel scaffold; baseline (speedup 1.0000x reference)
import jax
import jax.numpy as jnp
from jax.experimental import pallas as pl


def kernel(X, table):
    raise NotImplementedError("write your pallas kernel here")


# speedup vs baseline: 1.8176x; 1.8176x over previous
import functools

import jax
import jax.numpy as jnp
from jax import lax
from jax.experimental import pallas as pl
from jax.experimental.pallas import tpu as pltpu
from jax.experimental.pallas import tpu_sc as plsc

_INFO = plsc.get_sparse_core_info()
_NC, _NS = _INFO.num_cores, _INFO.num_subcores
_NW = _NC * _NS

_C = 8


@functools.partial(jax.jit, static_argnums=())
def _gather_rows(idx, table):
    (N,) = idx.shape
    V, D = table.shape
    b_per_w = N // _NW
    n_chunks = b_per_w // _C
    mesh = plsc.VectorSubcoreMesh(core_axis_name="c", subcore_axis_name="s")

    @functools.partial(
        pl.kernel,
        mesh=mesh,
        out_type=jax.ShapeDtypeStruct((N, D), jnp.float32),
        scratch_types=[
            pltpu.VMEM((b_per_w,), jnp.int32),
            pltpu.VMEM((_C, D), jnp.float32),
            pltpu.SemaphoreType.DMA,
        ],
    )
    def body(idx_hbm, table_hbm, out_hbm, idx_v, rows_v, sem):
        wid = lax.axis_index("s") * _NC + lax.axis_index("c")
        base = wid * b_per_w
        pltpu.sync_copy(idx_hbm.at[pl.ds(base, b_per_w)], idx_v)

        def chunk(j, carry):
            idx_chunk = idx_v.at[pl.ds(j * _C, _C)]
            pltpu.async_copy(table_hbm.at[idx_chunk], rows_v, sem).wait()
            pltpu.sync_copy(rows_v, out_hbm.at[pl.ds(base + j * _C, _C)])
            return carry

        lax.fori_loop(0, n_chunks, chunk, 0)

    return body(idx, table)


def kernel(X, table):
    B, T = X.shape
    idx = X.reshape(B * T).astype(jnp.int32)
    out = _gather_rows(idx, table)
    return out.reshape(B, T, table.shape[1])


# speedup vs baseline: 1.9439x; 1.0694x over previous
import functools

import jax
import jax.numpy as jnp
from jax import lax
from jax.experimental import pallas as pl
from jax.experimental.pallas import tpu as pltpu
from jax.experimental.pallas import tpu_sc as plsc

_INFO = plsc.get_sparse_core_info()
_NC, _NS = _INFO.num_cores, _INFO.num_subcores
_NW = _NC * _NS

_C = 4
_NBUF = 2


@jax.jit
def _gather_rows(idx2, table):
    n_rows_total, c = idx2.shape
    N = n_rows_total * c
    V, D = table.shape
    b_per_w = N // _NW
    n_chunks = b_per_w // _C
    n_steady = n_chunks // _NBUF - 1
    mesh = plsc.VectorSubcoreMesh(core_axis_name="c", subcore_axis_name="s")

    @functools.partial(
        pl.kernel,
        mesh=mesh,
        out_type=jax.ShapeDtypeStruct((N, D), jnp.float32),
        scratch_types=[
            pltpu.VMEM((n_chunks, _C), jnp.int32),
            pltpu.VMEM((_NBUF, _C, D), jnp.float32),
            pltpu.SemaphoreType.DMA,
            pltpu.SemaphoreType.DMA,
            pltpu.SemaphoreType.DMA,
            pltpu.SemaphoreType.DMA,
        ],
    )
    def body(idx_hbm, table_hbm, out_hbm, idx_v, rows_v, g0, g1, s0, s1):
        gsem = (g0, g1)
        ssem = (s0, s1)
        wid = lax.axis_index("s") * _NC + lax.axis_index("c")
        base = wid * b_per_w
        pltpu.sync_copy(idx_hbm.at[pl.ds(wid * n_chunks, n_chunks), :], idx_v)

        def gather(ch, b):
            pltpu.async_copy(table_hbm.at[idx_v.at[ch]], rows_v.at[b], gsem[b])

        def scatter(ch, b):
            pltpu.async_copy(
                rows_v.at[b], out_hbm.at[pl.ds(base + ch * _C, _C)], ssem[b])

        for b in range(_NBUF):
            gather(b, b)

        def step(s, carry):
            for b in range(_NBUF):
                ch = s * _NBUF + b
                pltpu.make_async_copy(
                    table_hbm.at[idx_v.at[ch]], rows_v.at[b], gsem[b]).wait()
                scatter(ch, b)
                pltpu.make_async_copy(
                    rows_v.at[b], out_hbm.at[pl.ds(base + ch * _C, _C)],
                    ssem[b]).wait()
                gather(ch + _NBUF, b)
            return carry

        lax.fori_loop(0, n_steady, step, 0)

        for b in range(_NBUF):
            ch = n_chunks - _NBUF + b
            pltpu.make_async_copy(
                table_hbm.at[idx_v.at[ch]], rows_v.at[b], gsem[b]).wait()
            scatter(ch, b)
        for b in range(_NBUF):
            ch = n_chunks - _NBUF + b
            pltpu.make_async_copy(
                rows_v.at[b], out_hbm.at[pl.ds(base + ch * _C, _C)],
                ssem[b]).wait()

    return body(idx2, table)


def kernel(X, table):
    B, T = X.shape
    idx2 = X.reshape(B * T // _C, _C).astype(jnp.int32)
    out = _gather_rows(idx2, table)
    return out.reshape(B, T, table.shape[1])
